# baseline (speedup 1.0000x reference)
import functools

import jax
import jax.numpy as jnp
from jax import lax
from jax.experimental import pallas as pl
from jax.experimental.pallas import tpu as pltpu
from jax.experimental.pallas import tpu_sc as plsc

B, S, H, MAXN = 16, 4096, 1024, 64

_NC, _NS, _L = 2, 16, 16
_NW = _NC * _NS
PAIRS = B * MAXN
PPW = PAIRS // _NW
RPW = 2 * PPW
ROWS = 2 * PAIRS


def _index_kernel(nm_ref, first_ref, last_ref, scale_ref):
    nm = nm_ref[0]
    ids = lax.broadcasted_iota(jnp.int32, (MAXN, 1), 0) + 1
    match = nm == ids
    pos = lax.broadcasted_iota(jnp.int32, (MAXN, S), 1)
    first = jnp.min(jnp.where(match, pos, S), axis=1)
    last = jnp.max(jnp.where(match, pos, -1), axis=1)
    present = last >= 0
    base = pl.program_id(0) * S
    first_ref[0, 0, :] = jnp.where(present, first, 0) + base
    last_ref[0, 0, :] = jnp.where(present, last, 0) + base
    scale_ref[0, 0, :] = present.astype(jnp.float32)


def _compute_indices(nm3):
    return pl.pallas_call(
        _index_kernel,
        grid=(B,),
        in_specs=[pl.BlockSpec((1, 1, S), lambda b: (b, 0, 0))],
        out_specs=[
            pl.BlockSpec((1, 1, MAXN), lambda b: (b, 0, 0)),
            pl.BlockSpec((1, 1, MAXN), lambda b: (b, 0, 0)),
            pl.BlockSpec((1, 1, MAXN), lambda b: (b, 0, 0)),
        ],
        out_shape=[
            jax.ShapeDtypeStruct((B, 1, MAXN), jnp.int32),
            jax.ShapeDtypeStruct((B, 1, MAXN), jnp.int32),
            jax.ShapeDtypeStruct((B, 1, MAXN), jnp.float32),
        ],
    )(nm3)


def _gather_body(table_hbm, first_hbm, last_hbm, scale_hbm, out_hbm,
                 fl_v, idx_v, scale_v, rows_v, sem):
    wid = lax.axis_index("s") * _NC + lax.axis_index("c")
    pbase = wid * PPW
    rbase = wid * RPW
    pltpu.sync_copy(first_hbm.at[pl.ds(pbase, PPW)], fl_v.at[pl.ds(0, PPW)])
    pltpu.sync_copy(last_hbm.at[pl.ds(pbase, PPW)], fl_v.at[pl.ds(PPW, PPW)])
    pltpu.sync_copy(scale_hbm.at[pl.ds(pbase, PPW)], scale_v)

    lane = lax.broadcasted_iota(jnp.int32, (_L,), 0)
    for g in range(RPW // _L):
        perm = (g * (_L // 2) + (lane >> 1)) + (lane & 1) * PPW
        idx_v[pl.ds(g * _L, _L)] = plsc.load_gather(fl_v, [perm])

    pltpu.async_copy(table_hbm.at[idx_v], rows_v, sem).wait()

    m = scale_v[pl.ds(0, _L)]
    for g in range(1, PPW // _L):
        m = jnp.minimum(m, scale_v[pl.ds(g * _L, _L)])
    all_present = jnp.min(m)

    @pl.when(all_present < 0.5)
    def _mask_rows():
        def col_body(c, carry):
            off = c * _L
            for r in range(RPW):
                srow = plsc.load_gather(
                    scale_v, [jnp.full((_L,), r // 2, jnp.int32)])
                rows_v[r, pl.ds(off, _L)] = rows_v[r, pl.ds(off, _L)] * srow
            return carry
        lax.fori_loop(0, H // _L, col_body, 0)

    pltpu.sync_copy(rows_v, out_hbm.at[pl.ds(rbase, RPW)])


@functools.cache
def _gather_rows():
    return pl.kernel(
        _gather_body,
        out_type=jax.ShapeDtypeStruct((ROWS, H), jnp.float32),
        mesh=plsc.VectorSubcoreMesh(core_axis_name="c", subcore_axis_name="s"),
        compiler_params=pltpu.CompilerParams(needs_layout_passes=False),
        scratch_types=[
            pltpu.VMEM((RPW,), jnp.int32),
            pltpu.VMEM((RPW,), jnp.int32),
            pltpu.VMEM((PPW,), jnp.float32),
            pltpu.VMEM((RPW, H), jnp.float32),
            pltpu.SemaphoreType.DMA,
        ],
    )


def kernel(input, attention_mask, question_mask, number_mask):
    nm3 = number_mask.astype(jnp.int32).reshape(B, 1, S)
    first, last, scale = _compute_indices(nm3)
    table = input.reshape(B * S, H)
    gathered = _gather_rows()(
        table, first.reshape(PAIRS), last.reshape(PAIRS),
        scale.reshape(PAIRS))
    return gathered.reshape(B, MAXN, 2 * H)


# speedup vs baseline: 1.6946x; 1.6946x over previous
import functools

import jax
import jax.numpy as jnp
from jax import lax
from jax.experimental import pallas as pl
from jax.experimental.pallas import tpu as pltpu
from jax.experimental.pallas import tpu_sc as plsc

B, S, H, MAXN = 16, 4096, 1024, 64

_NC, _NS, _L = 2, 16, 16
_NW = _NC * _NS
PAIRS = B * MAXN
PPW = PAIRS // _NW


def _index_kernel(nm_ref, first_ref, last_ref, scale_ref):
    ids = lax.broadcasted_iota(jnp.int32, (MAXN, 1), 0) + 1
    pos = lax.broadcasted_iota(jnp.int32, (MAXN, S), 1)
    for b in range(B):
        nm = nm_ref[pl.ds(b, 1), :]
        match = nm == ids
        first = jnp.min(jnp.where(match, pos, S), axis=1)
        last = jnp.max(jnp.where(match, pos, -1), axis=1)
        present = last >= 0
        sl = pl.ds(b * MAXN, MAXN)
        first_ref[sl] = jnp.where(present, first, 0) + b * S
        last_ref[sl] = jnp.where(present, last, 0) + b * S
        scale_ref[sl] = present.astype(jnp.float32)


def _compute_indices(nm):
    return pl.pallas_call(
        _index_kernel,
        out_shape=[
            jax.ShapeDtypeStruct((PAIRS,), jnp.int32),
            jax.ShapeDtypeStruct((PAIRS,), jnp.int32),
            jax.ShapeDtypeStruct((PAIRS,), jnp.float32),
        ],
    )(nm)


def _gather_body(table_hbm, first_hbm, last_hbm, scale_hbm, out_hbm,
                 fidx_v, lidx_v, scale_v, comb_v, sem):
    wid = lax.axis_index("s") * _NC + lax.axis_index("c")
    pbase = wid * PPW
    pltpu.sync_copy(first_hbm.at[pl.ds(pbase, PPW)], fidx_v)
    pltpu.sync_copy(last_hbm.at[pl.ds(pbase, PPW)], lidx_v)
    pltpu.sync_copy(scale_hbm.at[pl.ds(pbase, PPW)], scale_v)

    c1 = pltpu.async_copy(table_hbm.at[fidx_v], comb_v.at[:, pl.ds(0, H)], sem)
    c2 = pltpu.async_copy(table_hbm.at[lidx_v], comb_v.at[:, pl.ds(H, H)], sem)
    c1.wait()
    c2.wait()

    m = scale_v[pl.ds(0, _L)]
    for g in range(1, PPW // _L):
        m = jnp.minimum(m, scale_v[pl.ds(g * _L, _L)])
    all_present = jnp.min(m)

    @pl.when(all_present < 0.5)
    def _mask_rows():
        def col_body(c, carry):
            off = c * _L
            for r in range(PPW):
                srow = plsc.load_gather(
                    scale_v, [jnp.full((_L,), r, jnp.int32)])
                comb_v[r, pl.ds(off, _L)] = comb_v[r, pl.ds(off, _L)] * srow
            return carry
        lax.fori_loop(0, 2 * H // _L, col_body, 0)

    pltpu.sync_copy(comb_v, out_hbm.at[pl.ds(pbase, PPW)])


@functools.cache
def _gather_rows():
    return pl.kernel(
        _gather_body,
        out_type=jax.ShapeDtypeStruct((PAIRS, 2 * H), jnp.float32),
        mesh=plsc.VectorSubcoreMesh(core_axis_name="c", subcore_axis_name="s"),
        compiler_params=pltpu.CompilerParams(needs_layout_passes=False),
        scratch_types=[
            pltpu.VMEM((PPW,), jnp.int32),
            pltpu.VMEM((PPW,), jnp.int32),
            pltpu.VMEM((PPW,), jnp.float32),
            pltpu.VMEM((PPW, 2 * H), jnp.float32),
            pltpu.SemaphoreType.DMA,
        ],
    )


def kernel(input, attention_mask, question_mask, number_mask):
    nm = number_mask.astype(jnp.int32)
    first, last, scale = _compute_indices(nm)
    table = input.reshape(B * S, H)
    gathered = _gather_rows()(table, first, last, scale)
    return gathered.reshape(B, MAXN, 2 * H)
